# Initial kernel scaffold; baseline (speedup 1.0000x reference)
import jax
import jax.numpy as jnp
from jax.experimental import pallas as pl


def kernel(x, seg, tok_embed, pos_embed, seg_embed, gamma, beta):
    raise NotImplementedError("write your pallas kernel here")


# speedup vs baseline: 4.8781x; 4.8781x over previous
import functools

import jax
import jax.numpy as jnp
from jax import lax
from jax.experimental import pallas as pl
from jax.experimental.pallas import tpu as pltpu
from jax.experimental.pallas import tpu_sc as plsc

NC, NS, L = 2, 16, 16
NW = NC * NS
B, S, D = 1024, 200, 128
N = B * S
TPW = N // NW
C = 128
NCHUNK = TPW // C
NJ = D // L
EPS = 1e-5

_mesh = plsc.VectorSubcoreMesh(core_axis_name="c", subcore_axis_name="s")


def _rsqrt(v):
    y = lax.bitcast_convert_type(
        jnp.full((L,), 0x5F3759DF, jnp.int32)
        - (lax.bitcast_convert_type(v, jnp.int32) >> 1),
        jnp.float32,
    )
    for _ in range(3):
        y = y * (1.5 - 0.5 * v * y * y)
    return y


_DN = lax.GatherDimensionNumbers(
    offset_dims=(), collapsed_slice_dims=(0,), start_index_map=(0,))


def _gather16(vec, idx):
    return lax.gather(vec, idx[:, None], _DN, slice_sizes=(1,),
                      mode=lax.GatherScatterMode.PROMISE_IN_BOUNDS)


def _splat(vec, lane):
    return _gather16(vec, jnp.full((L,), lane, jnp.int32))


def _allsum(v, perms):
    for pm in perms:
        v = v + _gather16(v, pm)
    return v


def _body(x_ref, seg_ref, tok_ref, pos_ref, sege_ref, gam_ref, bet_ref, out_ref,
          idx_v, seg_v, pos2_v, buf_v, se_v, par_v, sem):
    wid = lax.axis_index("s") * NC + lax.axis_index("c")
    base_tok = wid * TPW

    pltpu.sync_copy(x_ref.at[pl.ds(base_tok, TPW)], idx_v)
    pltpu.sync_copy(seg_ref.at[pl.ds(base_tok, TPW)], seg_v)
    pltpu.sync_copy(pos_ref.at[pl.ds(0, S)], pos2_v)
    pltpu.sync_copy(sege_ref, se_v)
    pltpu.sync_copy(gam_ref, par_v.at[0])
    pltpu.sync_copy(bet_ref, par_v.at[1])

    s0 = [se_v[0, pl.ds(j * L, L)] for j in range(NJ)]
    s1 = [se_v[1, pl.ds(j * L, L)] for j in range(NJ)]
    delta = [s1[j] - s0[j] for j in range(NJ)]
    gam = [par_v[0, pl.ds(j * L, L)] for j in range(NJ)]
    bet = [par_v[1, pl.ds(j * L, L)] for j in range(NJ)]

    perms = [jnp.arange(L, dtype=jnp.int32) ^ k for k in (8, 4, 2, 1)]

    def _fold(p, carry):
        for j in range(NJ):
            pos2_v[p, pl.ds(j * L, L)] = pos2_v[p, pl.ds(j * L, L)] + s0[j]
        return carry
    lax.fori_loop(0, S, _fold, 0)

    def _chunk(c, carry):
        pltpu.async_copy(tok_ref.at[idx_v.at[pl.ds(c * C, C)]], buf_v, sem).wait()

        def _row(r, carry2):
            r_glob = c * C + r
            segv = seg_v[pl.ds((r_glob // 16) * 16, L)]
            g = _splat(segv, r_glob % 16).astype(jnp.float32)
            p = r_glob % S
            v = [buf_v[r, pl.ds(j * L, L)] + pos2_v[p, pl.ds(j * L, L)]
                 + g * delta[j] for j in range(NJ)]
            ssum = v[0]
            s2 = v[0] * v[0]
            for j in range(1, NJ):
                ssum = ssum + v[j]
                s2 = s2 + v[j] * v[j]
            tot = _allsum(ssum, perms)
            tot2 = _allsum(s2, perms)
            mean = tot * (1.0 / D)
            var = tot2 * (1.0 / D) - mean * mean
            rstd = _rsqrt(var + EPS)
            for j in range(NJ):
                buf_v[r, pl.ds(j * L, L)] = (v[j] - mean) * (rstd * gam[j]) + bet[j]
            return carry2

        lax.fori_loop(0, C, _row, 0)
        pltpu.sync_copy(buf_v, out_ref.at[pl.ds(base_tok + c * C, C)])
        return carry

    lax.fori_loop(0, NCHUNK, _chunk, 0)


_emb = functools.partial(
    pl.kernel,
    out_type=jax.ShapeDtypeStruct((N, D), jnp.float32),
    mesh=_mesh,
    scratch_types=[
        pltpu.VMEM((TPW,), jnp.int32),
        pltpu.VMEM((TPW,), jnp.int32),
        pltpu.VMEM((S, D), jnp.float32),
        pltpu.VMEM((C, D), jnp.float32),
        pltpu.VMEM((2, D), jnp.float32),
        pltpu.VMEM((2, D), jnp.float32),
        pltpu.SemaphoreType.DMA,
    ],
)(_body)


def kernel(x, seg, tok_embed, pos_embed, seg_embed, gamma, beta):
    x1 = x.reshape(N).astype(jnp.int32)
    seg1 = seg.reshape(N).astype(jnp.int32)
    out = _emb(x1, seg1, tok_embed, pos_embed, seg_embed, gamma, beta)
    return out.reshape(B, S, D)


# speedup vs baseline: 6.1332x; 1.2573x over previous
import functools

import jax
import jax.numpy as jnp
from jax import lax
from jax.experimental import pallas as pl
from jax.experimental.pallas import tpu as pltpu
from jax.experimental.pallas import tpu_sc as plsc

NC, NS, L = 2, 16, 16
NW = NC * NS
B, S, D = 1024, 200, 128
N = B * S
TPW = N // NW
C = 128
NCHUNK = TPW // C
NJ = D // L
EPS = 1e-5

_mesh = plsc.VectorSubcoreMesh(core_axis_name="c", subcore_axis_name="s")


def _rsqrt(v):
    y = lax.bitcast_convert_type(
        jnp.full((L,), 0x5F3759DF, jnp.int32)
        - (lax.bitcast_convert_type(v, jnp.int32) >> 1),
        jnp.float32,
    )
    for _ in range(2):
        y = y * (1.5 - 0.5 * v * y * y)
    return y


_DN = lax.GatherDimensionNumbers(
    offset_dims=(), collapsed_slice_dims=(0,), start_index_map=(0,))


def _gather16(vec, idx):
    return lax.gather(vec, idx[:, None], _DN, slice_sizes=(1,),
                      mode=lax.GatherScatterMode.PROMISE_IN_BOUNDS)


def _splat(vec, lane):
    return _gather16(vec, jnp.full((L,), lane, jnp.int32))


def _allsum(v, perms):
    for pm in perms:
        v = v + _gather16(v, pm)
    return v


def _body(x_ref, seg_ref, tok_ref, pos_ref, sege_ref, gam_ref, bet_ref, out_ref,
          idx_v, seg_v, pos2_v, buf0, buf1, se_v, par_v, sem0, sem1):
    wid = lax.axis_index("s") * NC + lax.axis_index("c")
    base_tok = wid * TPW

    pltpu.sync_copy(x_ref.at[pl.ds(base_tok, TPW)], idx_v)
    pltpu.sync_copy(seg_ref.at[pl.ds(base_tok, TPW)], seg_v)
    pltpu.sync_copy(pos_ref.at[pl.ds(0, S)], pos2_v)
    pltpu.sync_copy(sege_ref, se_v)
    pltpu.sync_copy(gam_ref, par_v.at[0])
    pltpu.sync_copy(bet_ref, par_v.at[1])

    s0 = [se_v[0, pl.ds(j * L, L)] for j in range(NJ)]
    s1 = [se_v[1, pl.ds(j * L, L)] for j in range(NJ)]
    delta = [s1[j] - s0[j] for j in range(NJ)]
    gam = [par_v[0, pl.ds(j * L, L)] for j in range(NJ)]
    bet = [par_v[1, pl.ds(j * L, L)] for j in range(NJ)]

    perms = [jnp.arange(L, dtype=jnp.int32) ^ k for k in (8, 4, 2, 1)]

    def _fold(p, carry):
        for j in range(NJ):
            pos2_v[p, pl.ds(j * L, L)] = pos2_v[p, pl.ds(j * L, L)] + s0[j]
        return carry
    lax.fori_loop(0, S, _fold, 0)

    def _start(c, buf, sem):
        pltpu.async_copy(tok_ref.at[idx_v.at[pl.ds(c * C, C)]], buf, sem)

    def _proc(c, buf, sem):
        pltpu.make_async_copy(
            tok_ref.at[idx_v.at[pl.ds(0, C)]], buf, sem).wait()

        def _row(r, carry2):
            r_glob = c * C + r
            segv = seg_v[pl.ds((r_glob // 16) * 16, L)]
            g = _splat(segv, r_glob % 16).astype(jnp.float32)
            p = r_glob % S
            v = [buf[r, pl.ds(j * L, L)] + pos2_v[p, pl.ds(j * L, L)]
                 + g * delta[j] for j in range(NJ)]
            ssum = v[0]
            s2 = v[0] * v[0]
            for j in range(1, NJ):
                ssum = ssum + v[j]
                s2 = s2 + v[j] * v[j]
            tot = _allsum(ssum, perms)
            tot2 = _allsum(s2, perms)
            mean = tot * (1.0 / D)
            var = tot2 * (1.0 / D) - mean * mean
            rstd = _rsqrt(var + EPS)
            for j in range(NJ):
                buf[r, pl.ds(j * L, L)] = (v[j] - mean) * (rstd * gam[j]) + bet[j]
            return carry2

        lax.fori_loop(0, C, _row, 0, unroll=2)
        pltpu.sync_copy(buf, out_ref.at[pl.ds(base_tok + c * C, C)])

    _start(0, buf0, sem0)

    def _chunk2(i, carry):
        c0 = i * 2
        _start(c0 + 1, buf1, sem1)
        _proc(c0, buf0, sem0)

        @pl.when(c0 + 2 < NCHUNK)
        def _prefetch():
            _start(c0 + 2, buf0, sem0)

        _proc(c0 + 1, buf1, sem1)
        return carry

    lax.fori_loop(0, NCHUNK // 2, _chunk2, 0)


_emb = functools.partial(
    pl.kernel,
    out_type=jax.ShapeDtypeStruct((N, D), jnp.float32),
    mesh=_mesh,
    scratch_types=[
        pltpu.VMEM((TPW,), jnp.int32),
        pltpu.VMEM((TPW,), jnp.int32),
        pltpu.VMEM((S, D), jnp.float32),
        pltpu.VMEM((C, D), jnp.float32),
        pltpu.VMEM((C, D), jnp.float32),
        pltpu.VMEM((2, D), jnp.float32),
        pltpu.VMEM((2, D), jnp.float32),
        pltpu.SemaphoreType.DMA,
        pltpu.SemaphoreType.DMA,
    ],
)(_body)


def kernel(x, seg, tok_embed, pos_embed, seg_embed, gamma, beta):
    x1 = x.reshape(N).astype(jnp.int32)
    seg1 = seg.reshape(N).astype(jnp.int32)
    out = _emb(x1, seg1, tok_embed, pos_embed, seg_embed, gamma, beta)
    return out.reshape(B, S, D)


# speedup vs baseline: 6.5011x; 1.0600x over previous
import functools

import jax
import jax.numpy as jnp
from jax import lax
from jax.experimental import pallas as pl
from jax.experimental.pallas import tpu as pltpu
from jax.experimental.pallas import tpu_sc as plsc

NC, NS, L = 2, 16, 16
NW = NC * NS
B, S, D = 1024, 200, 128
N = B * S
TPW = N // NW
C = 128
NCHUNK = TPW // C
NJ = D // L
NG = C // L
EPS = 1e-5

_mesh = plsc.VectorSubcoreMesh(core_axis_name="c", subcore_axis_name="s")


def _rsqrt(v):
    y = lax.bitcast_convert_type(
        jnp.full((L,), 0x5F3759DF, jnp.int32)
        - (lax.bitcast_convert_type(v, jnp.int32) >> 1),
        jnp.float32,
    )
    for _ in range(2):
        y = y * (1.5 - 0.5 * v * y * y)
    return y


_DN = lax.GatherDimensionNumbers(
    offset_dims=(), collapsed_slice_dims=(0,), start_index_map=(0,))


def _gather16(vec, idx):
    return lax.gather(vec, idx[:, None], _DN, slice_sizes=(1,),
                      mode=lax.GatherScatterMode.PROMISE_IN_BOUNDS)


def _splat(vec, lane):
    return _gather16(vec, jnp.full((L,), lane, jnp.int32))


def _allsum(v, perms):
    for pm in perms:
        v = v + _gather16(v, pm)
    return v


def _body(x_ref, seg_ref, tok_ref, pos_ref, sege_ref, gam_ref, bet_ref, out_ref,
          idx_v, seg_v, pos2_v, bufs0, bufs1, bufs2, se_v, par_v,
          gsem0, gsem1, gsem2, osem0, osem1, osem2):
    bufs = (bufs0, bufs1, bufs2)
    gsems = (gsem0, gsem1, gsem2)
    osems = (osem0, osem1, osem2)
    wid = lax.axis_index("s") * NC + lax.axis_index("c")
    base_tok = wid * TPW

    pltpu.sync_copy(x_ref.at[pl.ds(base_tok, TPW)], idx_v)
    pltpu.sync_copy(seg_ref.at[pl.ds(base_tok, TPW)], seg_v)
    pltpu.sync_copy(pos_ref.at[pl.ds(0, S)], pos2_v.at[pl.ds(0, S)])
    pltpu.sync_copy(pos_ref.at[pl.ds(0, S)], pos2_v.at[pl.ds(S, S)])
    pltpu.sync_copy(sege_ref, se_v)
    pltpu.sync_copy(gam_ref, par_v.at[0])
    pltpu.sync_copy(bet_ref, par_v.at[1])

    s0 = [se_v[0, pl.ds(j * L, L)] for j in range(NJ)]
    s1 = [se_v[1, pl.ds(j * L, L)] for j in range(NJ)]
    delta = [s1[j] - s0[j] for j in range(NJ)]
    gam = [par_v[0, pl.ds(j * L, L)] for j in range(NJ)]
    bet = [par_v[1, pl.ds(j * L, L)] for j in range(NJ)]

    perms = [jnp.arange(L, dtype=jnp.int32) ^ k for k in (8, 4, 2, 1)]

    def _fold(p, carry):
        for j in range(NJ):
            pos2_v[p, pl.ds(j * L, L)] = pos2_v[p, pl.ds(j * L, L)] + s0[j]
        return carry
    lax.fori_loop(0, 2 * S, _fold, 0)

    def _start(c, k):
        pltpu.async_copy(tok_ref.at[idx_v.at[pl.ds(c * C, C)]], bufs[k],
                         gsems[k])

    def _wait_out(k):
        pltpu.make_async_copy(
            bufs[k], out_ref.at[pl.ds(base_tok, C)], osems[k]).wait()

    def _proc(c, k):
        buf = bufs[k]
        pltpu.make_async_copy(
            tok_ref.at[idx_v.at[pl.ds(0, C)]], buf, gsems[k]).wait()
        pbase = (c * C) % S

        def _group(gi, carry2):
            r0 = gi * L
            segf = seg_v[pl.ds(c * C + r0, L)].astype(jnp.float32)
            for i in range(L):
                r = r0 + i
                g = _splat(segf, i)
                p = pbase + r
                v = [buf[r, pl.ds(j * L, L)] + pos2_v[p, pl.ds(j * L, L)]
                     + g * delta[j] for j in range(NJ)]
                ssum = v[0]
                s2 = v[0] * v[0]
                for j in range(1, NJ):
                    ssum = ssum + v[j]
                    s2 = s2 + v[j] * v[j]
                tot = _allsum(ssum, perms)
                tot2 = _allsum(s2, perms)
                mean = tot * (1.0 / D)
                var = tot2 * (1.0 / D) - mean * mean
                rstd = _rsqrt(var + EPS)
                for j in range(NJ):
                    buf[r, pl.ds(j * L, L)] = ((v[j] - mean) * (rstd * gam[j])
                                               + bet[j])
            return carry2

        lax.fori_loop(0, NG, _group, 0)
        pltpu.async_copy(buf, out_ref.at[pl.ds(base_tok + c * C, C)], osems[k])

    _start(0, 0)
    _start(1, 1)

    def _iter3(i, carry):
        c = 3 * i
        for u in range(3):
            cu = c + u
            _proc(cu, u)
            ku = (u + 2) % 3
            if u == 0:
                @pl.when(i > 0)
                def _w():
                    _wait_out(ku)
            else:
                _wait_out(ku)
            _start(cu + 2, ku)
        return carry

    lax.fori_loop(0, NCHUNK // 3, _iter3, 0)
    _proc(48, 0)
    _proc(49, 1)
    _wait_out(2)
    _wait_out(0)
    _wait_out(1)


_emb = functools.partial(
    pl.kernel,
    out_type=jax.ShapeDtypeStruct((N, D), jnp.float32),
    mesh=_mesh,
    scratch_types=[
        pltpu.VMEM((TPW,), jnp.int32),
        pltpu.VMEM((TPW,), jnp.int32),
        pltpu.VMEM((2 * S, D), jnp.float32),
        pltpu.VMEM((C, D), jnp.float32),
        pltpu.VMEM((C, D), jnp.float32),
        pltpu.VMEM((C, D), jnp.float32),
        pltpu.VMEM((2, D), jnp.float32),
        pltpu.VMEM((2, D), jnp.float32),
        pltpu.SemaphoreType.DMA,
        pltpu.SemaphoreType.DMA,
        pltpu.SemaphoreType.DMA,
        pltpu.SemaphoreType.DMA,
        pltpu.SemaphoreType.DMA,
        pltpu.SemaphoreType.DMA,
    ],
)(_body)


def kernel(x, seg, tok_embed, pos_embed, seg_embed, gamma, beta):
    x1 = x.reshape(N).astype(jnp.int32)
    seg1 = seg.reshape(N).astype(jnp.int32)
    out = _emb(x1, seg1, tok_embed, pos_embed, seg_embed, gamma, beta)
    return out.reshape(B, S, D)
